# baseline (speedup 1.0000x reference)
import jax
import jax.numpy as jnp
from jax import lax
from jax.experimental import pallas as pl
from jax.experimental.pallas import tpu as pltpu
from jax.experimental.pallas import tpu_sc as plsc

B, C, H, W = 16, 64, 128, 128
HW = H * W
K = 500
NW = 32
PAIRS = 256
NIDX = PAIRS * C
KHI = K - PAIRS
NQ = 4
QP = PAIRS // NQ
QIDX = NIDX // NQ


def _sc_loss_kernel(outflat, ind2d, mask_in, targ_in, out_hbm,
                    ind_v, idx_v, pred_v, mask_v, targ_v, out_v,
                    sem_g, sem_m, sem_t):
    cid = lax.axis_index("c")
    sid = lax.axis_index("s")
    row = cid * 16 + sid
    boff = sid * (C * HW)

    pltpu.sync_copy(ind2d.at[row], ind_v)

    @pl.when(cid == 0)
    def _():
        pltpu.async_copy(mask_in.at[sid, pl.ds(0, PAIRS)],
                         mask_v.at[pl.ds(0, PAIRS)], sem_m)
        pltpu.async_copy(targ_in.at[sid, pl.ds(0, PAIRS)],
                         targ_v.at[pl.ds(0, PAIRS)], sem_t)

    @pl.when(cid == 1)
    def _():
        pltpu.async_copy(mask_in.at[sid, pl.ds(PAIRS, KHI)],
                         mask_v.at[pl.ds(0, KHI)], sem_m)
        pltpu.async_copy(targ_in.at[sid, pl.ds(PAIRS, KHI)],
                         targ_v.at[pl.ds(0, KHI)], sem_t)

    lanes = lax.iota(jnp.int32, 16)
    coffs = [lanes * HW + (q * 16 * HW) for q in range(4)]

    def build(p, _):
        blk = (p // 16) * 16
        ivec = ind_v[pl.ds(blk, 16)] + boff
        av = ivec.at[lanes * 0 + (p - blk)].get(mode="promise_in_bounds")
        for q in range(4):
            idx_v[pl.ds(p * C + q * 16, 16)] = av + coffs[q]
        return 0

    for qq in range(NQ):
        lax.fori_loop(qq * QP, (qq + 1) * QP, build, 0)
        sl = pl.ds(qq * QIDX, QIDX)
        pltpu.async_copy(outflat.at[idx_v.at[sl]], pred_v.at[sl], sem_g)

    @pl.when(cid == 0)
    def _():
        pltpu.make_async_copy(mask_in.at[sid, pl.ds(0, PAIRS)],
                              mask_v.at[pl.ds(0, PAIRS)], sem_m).wait()
        pltpu.make_async_copy(targ_in.at[sid, pl.ds(0, PAIRS)],
                              targ_v.at[pl.ds(0, PAIRS)], sem_t).wait()

    @pl.when(cid == 1)
    def _():
        pltpu.make_async_copy(mask_in.at[sid, pl.ds(PAIRS, KHI)],
                              mask_v.at[pl.ds(0, KHI)], sem_m).wait()
        pltpu.make_async_copy(targ_in.at[sid, pl.ds(PAIRS, KHI)],
                              targ_v.at[pl.ds(0, KHI)], sem_t).wait()

    npairs = jnp.where(cid == 0, PAIRS, KHI)
    zero = jnp.zeros((16,), jnp.float32)

    def accum(r, carry):
        aabs, am = carry
        for q in range(4):
            sl = pl.ds(q * 16, 16)
            v = pred_v[pl.ds(r * C + q * 16, 16)]
            m = mask_v[r, sl]
            t = targ_v[r, sl]
            aabs = aabs + jnp.abs(v * m - t * m)
            am = am + m
        return (aabs, am)

    acc = (zero, zero)
    for qq in range(NQ):
        sl = pl.ds(qq * QIDX, QIDX)
        pltpu.make_async_copy(outflat.at[idx_v.at[sl]], pred_v.at[sl],
                              sem_g).wait()
        hi = jnp.minimum((qq + 1) * QP, npairs)
        acc = lax.fori_loop(qq * QP, hi, accum, acc)

    out_v[pl.ds(0, 16)] = acc[0]
    out_v[pl.ds(16, 16)] = acc[1]
    pltpu.sync_copy(out_v, out_hbm.at[row])


@jax.jit
def kernel(output, mask, ind, target):
    outflat = output.reshape(-1)
    lo = ind[:, :PAIRS]
    hi = jnp.pad(ind[:, PAIRS:], ((0, 0), (0, PAIRS - KHI)), mode="reflect")
    ind2d = jnp.concatenate([lo, hi], axis=0)

    mesh = plsc.VectorSubcoreMesh(core_axis_name="c", subcore_axis_name="s")
    partials = pl.kernel(
        _sc_loss_kernel,
        mesh=mesh,
        out_type=jax.ShapeDtypeStruct((NW, 32), jnp.float32),
        scratch_types=[
            pltpu.VMEM((PAIRS,), jnp.int32),
            pltpu.VMEM((NIDX,), jnp.int32),
            pltpu.VMEM((NIDX,), jnp.float32),
            pltpu.VMEM((PAIRS, C), jnp.float32),
            pltpu.VMEM((PAIRS, C), jnp.float32),
            pltpu.VMEM((32,), jnp.float32),
            pltpu.SemaphoreType.DMA,
            pltpu.SemaphoreType.DMA,
            pltpu.SemaphoreType.DMA,
        ],
    )(outflat, ind2d, mask, target)

    loss = jnp.sum(partials[:, :16]) / (jnp.sum(partials[:, 16:]) + 0.0001)
    return loss


# speedup vs baseline: 1.0268x; 1.0268x over previous
import jax
import jax.numpy as jnp
from jax import lax
from jax.experimental import pallas as pl
from jax.experimental.pallas import tpu as pltpu
from jax.experimental.pallas import tpu_sc as plsc

B, C, H, W = 16, 64, 128, 128
HW = H * W
K = 500
NW = 32
PAIRS = 256
NIDX = PAIRS * C
KHI = K - PAIRS
NQ = 4
QP = PAIRS // NQ
QIDX = NIDX // NQ


def _sc_loss_kernel(outflat, ind2d, mask_in, targ_in, out_hbm,
                    ind_v, idx_v, pred_v, mask_v, targ_v, out_v,
                    sem_g, sem_m, sem_t):
    cid = lax.axis_index("c")
    sid = lax.axis_index("s")
    row = cid * 16 + sid
    boff = sid * (C * HW)

    pltpu.sync_copy(ind2d.at[row], ind_v)

    @pl.when(cid == 0)
    def _():
        pltpu.async_copy(mask_in.at[sid, pl.ds(0, PAIRS)],
                         mask_v.at[pl.ds(0, PAIRS)], sem_m)
        pltpu.async_copy(targ_in.at[sid, pl.ds(0, PAIRS)],
                         targ_v.at[pl.ds(0, PAIRS)], sem_t)

    @pl.when(cid == 1)
    def _():
        pltpu.async_copy(mask_in.at[sid, pl.ds(PAIRS, KHI)],
                         mask_v.at[pl.ds(0, KHI)], sem_m)
        pltpu.async_copy(targ_in.at[sid, pl.ds(PAIRS, KHI)],
                         targ_v.at[pl.ds(0, KHI)], sem_t)

    lanes = lax.iota(jnp.int32, 16)
    coffs = [lanes * HW + (q * 16 * HW) for q in range(4)]

    def build(p, _):
        blk = (p // 16) * 16
        ivec = ind_v[pl.ds(blk, 16)] + boff
        av = ivec.at[lanes * 0 + (p - blk)].get(mode="promise_in_bounds")
        for q in range(4):
            idx_v[pl.ds(p * C + q * 16, 16)] = av + coffs[q]
        return 0

    for qq in range(NQ):
        lax.fori_loop(qq * QP, (qq + 1) * QP, build, 0)
        sl = pl.ds(qq * QIDX, QIDX)
        pltpu.async_copy(outflat.at[idx_v.at[sl]], pred_v.at[sl], sem_g)

    @pl.when(cid == 0)
    def _():
        pltpu.make_async_copy(mask_in.at[sid, pl.ds(0, PAIRS)],
                              mask_v.at[pl.ds(0, PAIRS)], sem_m).wait()
        pltpu.make_async_copy(targ_in.at[sid, pl.ds(0, PAIRS)],
                              targ_v.at[pl.ds(0, PAIRS)], sem_t).wait()

    @pl.when(cid == 1)
    def _():
        pltpu.make_async_copy(mask_in.at[sid, pl.ds(PAIRS, KHI)],
                              mask_v.at[pl.ds(0, KHI)], sem_m).wait()
        pltpu.make_async_copy(targ_in.at[sid, pl.ds(PAIRS, KHI)],
                              targ_v.at[pl.ds(0, KHI)], sem_t).wait()

    npairs = jnp.where(cid == 0, PAIRS, KHI)
    zero = jnp.zeros((16,), jnp.float32)

    def accum(r, carry):
        aabs, am = carry
        for q in range(4):
            sl = pl.ds(q * 16, 16)
            v = pred_v[pl.ds(r * C + q * 16, 16)]
            m = mask_v[r, sl]
            t = targ_v[r, sl]
            aabs = aabs + jnp.abs(v * m - t * m)
            am = am + m
        return (aabs, am)

    acc = (zero, zero)
    for qq in range(NQ):
        sl = pl.ds(qq * QIDX, QIDX)
        pltpu.make_async_copy(outflat.at[idx_v.at[sl]], pred_v.at[sl],
                              sem_g).wait()
        hi = jnp.minimum((qq + 1) * QP, npairs)
        acc = lax.fori_loop(qq * QP, hi, accum, acc)

    out_v[pl.ds(0, 16)] = acc[0]
    out_v[pl.ds(16, 16)] = acc[1]
    pltpu.sync_copy(out_v, out_hbm.at[row])


@jax.jit
def kernel(output, mask, ind, target):
    outflat = output.reshape(-1)
    lo = ind[:, :PAIRS]
    hi = jnp.pad(ind[:, PAIRS:], ((0, 0), (0, PAIRS - KHI)))
    ind2d = jnp.concatenate([lo, hi], axis=0)

    mesh = plsc.VectorSubcoreMesh(core_axis_name="c", subcore_axis_name="s")
    partials = pl.kernel(
        _sc_loss_kernel,
        mesh=mesh,
        out_type=jax.ShapeDtypeStruct((NW, 32), jnp.float32),
        scratch_types=[
            pltpu.VMEM((PAIRS,), jnp.int32),
            pltpu.VMEM((NIDX,), jnp.int32),
            pltpu.VMEM((NIDX,), jnp.float32),
            pltpu.VMEM((PAIRS, C), jnp.float32),
            pltpu.VMEM((PAIRS, C), jnp.float32),
            pltpu.VMEM((32,), jnp.float32),
            pltpu.SemaphoreType.DMA,
            pltpu.SemaphoreType.DMA,
            pltpu.SemaphoreType.DMA,
        ],
    )(outflat, ind2d, mask, target)

    loss = jnp.sum(partials[:, :16]) / (jnp.sum(partials[:, 16:]) + 0.0001)
    return loss
